# Initial kernel scaffold; baseline (speedup 1.0000x reference)
import jax
import jax.numpy as jnp
from jax.experimental import pallas as pl


def kernel(input_ids, base_table, soft_embeddings):
    raise NotImplementedError("write your pallas kernel here")


# speedup vs baseline: 1.8827x; 1.8827x over previous
import functools

import jax
import jax.numpy as jnp
from jax import lax
from jax.experimental import pallas as pl
from jax.experimental.pallas import tpu as pltpu
from jax.experimental.pallas import tpu_sc as plsc

VOCAB = 32128
DIM = 512
PH_START = 32100
PH_END = 32120
N_SOFT = PH_END - PH_START

B_TOTAL = 4 * 4096
NUM_WORKERS = 32
B_PER_W = B_TOTAL // NUM_WORKERS
CHUNK = 64
N_CHUNKS = B_PER_W // CHUNK


def _sc_lookup(ids, base_table, soft):
    mesh = plsc.VectorSubcoreMesh(core_axis_name="c", subcore_axis_name="s")

    @functools.partial(
        pl.kernel,
        mesh=mesh,
        out_type=jax.ShapeDtypeStruct((B_TOTAL, DIM), jnp.float32),
        scratch_types=[
            pltpu.VMEM((N_CHUNKS, CHUNK), jnp.int32),
            pltpu.VMEM_SHARED((N_SOFT, DIM), jnp.float32),
            pltpu.VMEM((CHUNK, DIM), jnp.float32),
            pltpu.SemaphoreType.DMA,
        ],
    )
    def body(ids_hbm, base_hbm, soft_hbm, out_hbm, idx_v, soft_sp, buf, sem):
        sid = lax.axis_index("s")
        wid = sid * 2 + lax.axis_index("c")
        row0 = wid * B_PER_W
        pltpu.sync_copy(ids_hbm.at[wid], idx_v)

        @pl.when(sid == 0)
        def _():
            pltpu.sync_copy(soft_hbm, soft_sp)

        plsc.subcore_barrier()

        def do_chunk(c, carry):
            pltpu.async_copy(base_hbm.at[idx_v.at[c]], buf, sem).wait()

            for v in range(CHUNK // 16):
                ids16 = idx_v[c, pl.ds(v * 16, 16)]
                for l in range(16):
                    tok = ids16[l]
                    is_ph = (tok >= PH_START) & (tok < PH_END)

                    @pl.when(is_ph)
                    def _(tok=tok, l=l, v=v):
                        pltpu.sync_copy(
                            soft_sp.at[tok - PH_START], buf.at[v * 16 + l]
                        )

            pltpu.sync_copy(buf, out_hbm.at[pl.ds(row0 + c * CHUNK, CHUNK)])
            return carry

        lax.fori_loop(0, N_CHUNKS, do_chunk, 0)

    return body(ids, base_table, soft)


def kernel(input_ids, base_table, soft_embeddings):
    ids = input_ids.reshape(NUM_WORKERS, N_CHUNKS, CHUNK).astype(jnp.int32)
    out = _sc_lookup(ids, base_table, soft_embeddings)
    return out.reshape(4, 4096, DIM)


# speedup vs baseline: 2.0139x; 1.0696x over previous
import functools

import jax
import jax.numpy as jnp
from jax import lax
from jax.experimental import pallas as pl
from jax.experimental.pallas import tpu as pltpu
from jax.experimental.pallas import tpu_sc as plsc

VOCAB = 32128
DIM = 512
PH_START = 32100
PH_END = 32120
N_SOFT = PH_END - PH_START

B_TOTAL = 4 * 4096
NUM_WORKERS = 32
B_PER_W = B_TOTAL // NUM_WORKERS
CHUNK = 64
N_CHUNKS = B_PER_W // CHUNK


def _sc_lookup(ids, base_table, soft):
    mesh = plsc.VectorSubcoreMesh(core_axis_name="c", subcore_axis_name="s")

    @functools.partial(
        pl.kernel,
        mesh=mesh,
        out_type=jax.ShapeDtypeStruct((B_TOTAL, DIM), jnp.float32),
        scratch_types=[
            pltpu.VMEM((N_CHUNKS, CHUNK), jnp.int32),
            pltpu.VMEM((CHUNK, DIM), jnp.float32),
            pltpu.VMEM((CHUNK, DIM), jnp.float32),
            pltpu.VMEM((CHUNK, DIM), jnp.float32),
            pltpu.SemaphoreType.DMA,
            pltpu.SemaphoreType.DMA,
            pltpu.SemaphoreType.DMA,
            pltpu.SemaphoreType.DMA,
            pltpu.SemaphoreType.DMA,
            pltpu.SemaphoreType.DMA,
        ],
    )
    def body(ids_hbm, base_hbm, soft_hbm, out_hbm, idx_v,
             buf0, buf1, buf2, sg0, sg1, sg2, ss0, ss1, ss2):
        sid = lax.axis_index("s")
        wid = sid * 2 + lax.axis_index("c")
        row0 = wid * B_PER_W
        bufs = (buf0, buf1, buf2)
        gsems = (sg0, sg1, sg2)
        ssems = (ss0, ss1, ss2)
        pltpu.sync_copy(ids_hbm.at[wid], idx_v)

        def patch(c, b):
            def patch_v(v, carry):
                ids16 = idx_v[c, pl.ds(v * 16, 16)]
                for l in range(16):
                    tok = ids16[l]
                    is_ph = (tok >= PH_START) & (tok < PH_END)

                    @pl.when(is_ph)
                    def _(tok=tok, l=l, v=v):
                        pltpu.sync_copy(
                            soft_hbm.at[tok - PH_START],
                            bufs[b].at[v * 16 + l],
                        )

                return carry

            lax.fori_loop(0, CHUNK // 16, patch_v, 0)

        def out_slice(c):
            return out_hbm.at[pl.ds(row0 + c * CHUNK, CHUNK)]

        gh = [None] * N_CHUNKS
        sh = [None] * N_CHUNKS
        gh[0] = pltpu.async_copy(base_hbm.at[idx_v.at[0]], bufs[0], gsems[0])
        for c in range(N_CHUNKS):
            b = c % 2
            nb = (c + 1) % 2
            gh[c].wait()
            if c - 1 >= 0:
                sh[c - 1].wait()
            patch(c, b)
            if c + 1 < N_CHUNKS:
                gh[c + 1] = pltpu.async_copy(
                    base_hbm.at[idx_v.at[c + 1]], bufs[nb], gsems[nb]
                )
            sh[c] = pltpu.async_copy(bufs[b], out_slice(c), ssems[b])
        sh[N_CHUNKS - 1].wait()

    return body(ids, base_table, soft)


def kernel(input_ids, base_table, soft_embeddings):
    ids = input_ids.reshape(NUM_WORKERS, N_CHUNKS, CHUNK).astype(jnp.int32)
    out = _sc_lookup(ids, base_table, soft_embeddings)
    return out.reshape(4, 4096, DIM)
